# baseline (speedup 1.0000x reference)
import functools

import jax
import jax.numpy as jnp
from jax import lax
from jax.experimental import pallas as pl
from jax.experimental.pallas import tpu as pltpu

try:
    from jax.experimental.pallas import tpu_sc as plsc
    _HAS_SC = True
except ImportError:
    plsc = None
    _HAS_SC = False

_CUTOFF = 5.0
_NUM_BASIS = 16
_EMBED_DIM = 8
_NW = 32


_CHUNK = 128


@functools.lru_cache(maxsize=None)
def _make_node_gather(b_pad: int, d: int):
    b_per_w = b_pad // _NW
    n_chunks = b_per_w // _CHUNK
    mesh = plsc.VectorSubcoreMesh(core_axis_name="c", subcore_axis_name="s")

    @functools.partial(
        pl.kernel,
        mesh=mesh,
        out_type=jax.ShapeDtypeStruct((b_pad, 128), jnp.float32),
        scratch_types=[
            pltpu.VMEM((n_chunks, _CHUNK), jnp.int32),
            pltpu.VMEM((3, _CHUNK, 128), jnp.float32),
            pltpu.SemaphoreType.DMA,
            pltpu.SemaphoreType.DMA,
            pltpu.SemaphoreType.DMA,
            pltpu.SemaphoreType.DMA,
            pltpu.SemaphoreType.DMA,
            pltpu.SemaphoreType.DMA,
        ],
    )
    def gather_kernel(idx_hbm, table_hbm, out_hbm, idx_v, rows_v,
                      g0, g1, g2, w0, w1, w2):
        wid = lax.axis_index("s") * 2 + lax.axis_index("c")
        pltpu.sync_copy(idx_hbm.at[wid], idx_v)
        gsems = (g0, g1, g2)
        wsems = (w0, w1, w2)

        def gather(c, b):
            pltpu.async_copy(table_hbm.at[idx_v.at[c]], rows_v.at[b], gsems[b])

        gather(0, 0)
        gather(1, 1)

        def step(i, carry):
            for b0 in range(3):
                c = i * 3 + b0
                b = b0

                @pl.when(c < n_chunks)
                def _():
                    pltpu.make_async_copy(
                        table_hbm.at[idx_v.at[c]], rows_v.at[b], gsems[b]
                    ).wait()
                    dst = wid * b_per_w + c * _CHUNK
                    pltpu.async_copy(
                        rows_v.at[b],
                        out_hbm.at[pl.ds(dst, _CHUNK)],
                        wsems[b],
                    )

                    nb = (b0 + 2) % 3

                    @pl.when(c + 2 < n_chunks)
                    def _():
                        @pl.when(c >= 1)
                        def _():
                            dst2 = wid * b_per_w + (c - 1) * _CHUNK
                            pltpu.make_async_copy(
                                rows_v.at[nb],
                                out_hbm.at[pl.ds(dst2, _CHUNK)],
                                wsems[nb],
                            ).wait()

                        gather(c + 2, nb)

            return carry

        lax.fori_loop(0, (n_chunks + 2) // 3, step, 0)

        for tail in (n_chunks - 3, n_chunks - 2, n_chunks - 1):
            b = tail % 3
            dst = wid * b_per_w + tail * _CHUNK
            pltpu.make_async_copy(
                rows_v.at[b], out_hbm.at[pl.ds(dst, _CHUNK)], wsems[b]
            ).wait()

    return gather_kernel


def _edge_body(a_ref, o_ref):
    ax = a_ref[0]
    ay = a_ref[1]
    az = a_ref[2]
    r2 = ax * ax + ay * ay + az * az
    rinv = lax.rsqrt(jnp.maximum(r2, 1e-18))
    r = r2 * rinv
    t = r * (jnp.pi / _CUTOFF)
    w = jnp.sqrt(2.0 / _CUTOFF) * rinv
    s1 = jnp.sin(t)
    d = 2.0 * jnp.cos(t)
    u_prev = w * s1
    u_cur = d * u_prev
    us = [u_prev, u_cur]
    for _ in range(_NUM_BASIS - 2):
        u_next = d * u_cur - u_prev
        u_prev, u_cur = u_cur, u_next
        us.append(u_cur)
    o_ref[...] = jnp.stack(us, axis=0)


def _edge_expand(edge_attr_t, n_edges: int, rb: int = 1000, interpret: bool = False):
    rows = n_edges // 128
    grid = rows // rb
    return pl.pallas_call(
        _edge_body,
        grid=(grid,),
        in_specs=[pl.BlockSpec((3, rb, 128), lambda i: (0, i, 0))],
        out_specs=pl.BlockSpec((_NUM_BASIS, rb, 128), lambda i: (0, i, 0)),
        out_shape=jax.ShapeDtypeStruct((_NUM_BASIS, rows, 128), jnp.float32),
        interpret=interpret,
    )(edge_attr_t)


def kernel(x, edge_attr, W_x, W_z):
    n_nodes = x.shape[0]
    d = 2 * _EMBED_DIM
    table = jnp.concatenate(
        [W_x, W_z, jnp.zeros((W_x.shape[0], 128 - d), jnp.float32)], axis=1
    )

    quantum = _NW * _CHUNK
    b_pad = ((n_nodes + quantum - 1) // quantum) * quantum
    idx = jnp.zeros((b_pad,), jnp.int32).at[:n_nodes].set(x.astype(jnp.int32))
    idx = idx.reshape(_NW, b_pad // (_NW * _CHUNK), _CHUNK)
    out_rows = _make_node_gather(b_pad, d)(idx, table)
    h_node_x = out_rows[:n_nodes, :_EMBED_DIM]
    h_node_z = out_rows[:n_nodes, _EMBED_DIM:d]

    n_edges = edge_attr.shape[0]
    ea_t = jnp.transpose(edge_attr).reshape(3, n_edges // 128, 128)
    out3 = _edge_expand(ea_t, n_edges)
    h_edge = out3.transpose(1, 2, 0).reshape(n_edges, _NUM_BASIS)
    return (h_node_x, h_node_z, h_edge)


# speedup vs baseline: 1.0074x; 1.0074x over previous
import functools

import jax
import jax.numpy as jnp
from jax import lax
from jax.experimental import pallas as pl
from jax.experimental.pallas import tpu as pltpu

try:
    from jax.experimental.pallas import tpu_sc as plsc
    _HAS_SC = True
except ImportError:
    plsc = None
    _HAS_SC = False

_CUTOFF = 5.0
_NUM_BASIS = 16
_EMBED_DIM = 8
_NW = 32


_CHUNK = 128


@functools.lru_cache(maxsize=None)
def _make_node_gather(b_pad: int, d: int):
    b_per_w = b_pad // _NW
    n_chunks = b_per_w // _CHUNK
    mesh = plsc.VectorSubcoreMesh(core_axis_name="c", subcore_axis_name="s")

    @functools.partial(
        pl.kernel,
        mesh=mesh,
        out_type=jax.ShapeDtypeStruct((b_pad, 128), jnp.float32),
        scratch_types=[
            pltpu.VMEM((n_chunks, _CHUNK), jnp.int32),
            pltpu.VMEM((3, _CHUNK, 128), jnp.float32),
            pltpu.SemaphoreType.DMA,
            pltpu.SemaphoreType.DMA,
            pltpu.SemaphoreType.DMA,
            pltpu.SemaphoreType.DMA,
            pltpu.SemaphoreType.DMA,
            pltpu.SemaphoreType.DMA,
        ],
    )
    def gather_kernel(idx_hbm, table_hbm, out_hbm, idx_v, rows_v,
                      g0, g1, g2, w0, w1, w2):
        wid = lax.axis_index("s") * 2 + lax.axis_index("c")
        pltpu.sync_copy(idx_hbm.at[wid], idx_v)
        gsems = (g0, g1, g2)
        wsems = (w0, w1, w2)

        def gather(c, b):
            pltpu.async_copy(table_hbm.at[idx_v.at[c]], rows_v.at[b], gsems[b])

        gather(0, 0)
        gather(1, 1)

        def step(i, carry):
            for b0 in range(3):
                c = i * 3 + b0
                b = b0

                @pl.when(c < n_chunks)
                def _():
                    pltpu.make_async_copy(
                        table_hbm.at[idx_v.at[c]], rows_v.at[b], gsems[b]
                    ).wait()
                    dst = wid * b_per_w + c * _CHUNK
                    pltpu.async_copy(
                        rows_v.at[b],
                        out_hbm.at[pl.ds(dst, _CHUNK)],
                        wsems[b],
                    )

                    nb = (b0 + 2) % 3

                    @pl.when(c + 2 < n_chunks)
                    def _():
                        @pl.when(c >= 1)
                        def _():
                            dst2 = wid * b_per_w + (c - 1) * _CHUNK
                            pltpu.make_async_copy(
                                rows_v.at[nb],
                                out_hbm.at[pl.ds(dst2, _CHUNK)],
                                wsems[nb],
                            ).wait()

                        gather(c + 2, nb)

            return carry

        lax.fori_loop(0, (n_chunks + 2) // 3, step, 0)

        for tail in (n_chunks - 3, n_chunks - 2, n_chunks - 1):
            b = tail % 3
            dst = wid * b_per_w + tail * _CHUNK
            pltpu.make_async_copy(
                rows_v.at[b], out_hbm.at[pl.ds(dst, _CHUNK)], wsems[b]
            ).wait()

    return gather_kernel


def _edge_body(a_ref, o_ref):
    ax = a_ref[0]
    ay = a_ref[1]
    az = a_ref[2]
    r2 = ax * ax + ay * ay + az * az
    r = jnp.sqrt(r2)
    t = r * (jnp.pi / _CUTOFF)
    w = jnp.sqrt(2.0 / _CUTOFF) / jnp.maximum(r, 1e-9)
    s1 = jnp.sin(t)
    d = 2.0 * jnp.cos(t)
    u_prev = w * s1
    u_cur = d * u_prev
    us = [u_prev, u_cur]
    for _ in range(_NUM_BASIS - 2):
        u_next = d * u_cur - u_prev
        u_prev, u_cur = u_cur, u_next
        us.append(u_cur)
    o_ref[...] = jnp.stack(us, axis=0)


def _edge_expand(edge_attr_t, n_edges: int, rb: int = 1000, interpret: bool = False):
    rows = n_edges // 128
    grid = rows // rb
    return pl.pallas_call(
        _edge_body,
        grid=(grid,),
        in_specs=[pl.BlockSpec((3, rb, 128), lambda i: (0, i, 0))],
        out_specs=pl.BlockSpec((_NUM_BASIS, rb, 128), lambda i: (0, i, 0)),
        out_shape=jax.ShapeDtypeStruct((_NUM_BASIS, rows, 128), jnp.float32),
        interpret=interpret,
    )(edge_attr_t)


def kernel(x, edge_attr, W_x, W_z):
    n_nodes = x.shape[0]
    d = 2 * _EMBED_DIM
    table = jnp.concatenate(
        [W_x, W_z, jnp.zeros((W_x.shape[0], 128 - d), jnp.float32)], axis=1
    )

    quantum = _NW * _CHUNK
    b_pad = ((n_nodes + quantum - 1) // quantum) * quantum
    idx = jnp.zeros((b_pad,), jnp.int32).at[:n_nodes].set(x.astype(jnp.int32))
    idx = idx.reshape(_NW, b_pad // (_NW * _CHUNK), _CHUNK)
    out_rows = _make_node_gather(b_pad, d)(idx, table)
    h_node_x = out_rows[:n_nodes, :_EMBED_DIM]
    h_node_z = out_rows[:n_nodes, _EMBED_DIM:d]

    n_edges = edge_attr.shape[0]
    ea_t = jnp.transpose(edge_attr).reshape(3, n_edges // 128, 128)
    out3 = _edge_expand(ea_t, n_edges)
    h_edge = out3.transpose(1, 2, 0).reshape(n_edges, _NUM_BASIS)
    return (h_node_x, h_node_z, h_edge)
